# Initial kernel scaffold; baseline (speedup 1.0000x reference)
import jax
import jax.numpy as jnp
from jax.experimental import pallas as pl


def kernel(x, edge_index, edge_mask, W, eps):
    raise NotImplementedError("write your pallas kernel here")


# speedup vs baseline: 5.2144x; 5.2144x over previous
import functools

import jax
import jax.numpy as jnp
from jax import lax
from jax.experimental import pallas as pl
from jax.experimental.pallas import tpu as pltpu
from jax.experimental.pallas import tpu_sc as plsc

N_NODES = 10000
N_EDGES = 320000
D = 128

NC = 2
NS = 16
NW = NC * NS
CHUNK = 128
NCHUNKS = N_EDGES // CHUNK
ITERS = -(-NCHUNKS // NW)
ROWS_PER_SUB = 624
TAIL_BASE = ROWS_PER_SUB * NS
TAIL = N_NODES - TAIL_BASE
ZSIZES = (128, 128, 128, 128, 112)


def _sc_segment_partials(x, src, dst, mask):
    mesh = plsc.VectorSubcoreMesh(
        core_axis_name="c", subcore_axis_name="s", num_cores=NC, num_subcores=NS
    )

    @functools.partial(
        pl.kernel,
        out_type=jax.ShapeDtypeStruct((NC, N_NODES, D), jnp.float32),
        mesh=mesh,
        scratch_types=[
            pltpu.VMEM((CHUNK,), jnp.int32),
            pltpu.VMEM((CHUNK,), jnp.int32),
            pltpu.VMEM((CHUNK,), jnp.float32),
            pltpu.VMEM((CHUNK, D), jnp.float32),
            pltpu.VMEM_SHARED((N_NODES, D), jnp.float32),
            pltpu.SemaphoreType.DMA,
        ],
    )
    def body(x_hbm, src_hbm, dst_hbm, mask_hbm, out_hbm,
             src_v, dst_v, mask_v, rows_v, accum, sem):
        cid = lax.axis_index("c")
        sid = lax.axis_index("s")
        wid = sid * NC + cid

        zeros16 = jnp.zeros((16,), jnp.float32)

        def zero_row(r, _):
            for j in range(D // 16):
                rows_v[r, pl.ds(j * 16, 16)] = zeros16
            return _

        lax.fori_loop(0, CHUNK, zero_row, None)
        my_base = pl.multiple_of(sid * ROWS_PER_SUB, 8)
        off = 0
        for zsz in ZSIZES:
            pltpu.sync_copy(
                rows_v.at[pl.ds(0, zsz)],
                accum.at[pl.ds(pl.multiple_of(my_base + off, 8), zsz)],
            )
            off += zsz

        @pl.when(sid == 0)
        def _():
            pltpu.sync_copy(rows_v.at[pl.ds(0, TAIL)], accum.at[pl.ds(TAIL_BASE, TAIL)])

        plsc.subcore_barrier()

        def scale_group(g, _):
            mvec = mask_v[pl.ds(g * 16, 16)]
            for j in range(16):
                m = jnp.full((16,), mvec[j], jnp.float32)
                e = g * 16 + j
                for k in range(D // 16):
                    rows_v[e, pl.ds(k * 16, 16)] = rows_v[e, pl.ds(k * 16, 16)] * m
            return _

        def chunk_body(i, _):
            chunk = i * NW + wid

            @pl.when(chunk < NCHUNKS)
            def _():
                base = pl.multiple_of(chunk * CHUNK, CHUNK)
                pltpu.sync_copy(src_hbm.at[pl.ds(base, CHUNK)], src_v)
                pltpu.sync_copy(dst_hbm.at[pl.ds(base, CHUNK)], dst_v)
                pltpu.sync_copy(mask_hbm.at[pl.ds(base, CHUNK)], mask_v)
                pltpu.async_copy(x_hbm.at[src_v], rows_v, sem).wait()
                lax.fori_loop(0, CHUNK // 16, scale_group, None)
                pltpu.sync_copy(rows_v, accum.at[dst_v], add=True)

            return _

        lax.fori_loop(0, ITERS, chunk_body, None)
        plsc.subcore_barrier()

        pltpu.sync_copy(
            accum.at[pl.ds(my_base, ROWS_PER_SUB)],
            out_hbm.at[cid, pl.ds(my_base, ROWS_PER_SUB)],
        )

        @pl.when(sid == 0)
        def _():
            pltpu.sync_copy(
                accum.at[pl.ds(TAIL_BASE, TAIL)],
                out_hbm.at[cid, pl.ds(TAIL_BASE, TAIL)],
            )

    return body(x, src, dst, mask)


def _tc_finish(x, p0, p1, wt, eps):
    BR = 1000

    def body(eps_ref, x_ref, p0_ref, p1_ref, wt_ref, out_ref):
        h = (1.0 + eps_ref[0]) * x_ref[...] + p0_ref[...] + p1_ref[...]
        out_ref[...] = jnp.dot(h, wt_ref[...], preferred_element_type=jnp.float32)

    return pl.pallas_call(
        body,
        grid=(N_NODES // BR,),
        in_specs=[
            pl.BlockSpec(memory_space=pltpu.SMEM),
            pl.BlockSpec((BR, D), lambda i: (i, 0)),
            pl.BlockSpec((BR, D), lambda i: (i, 0)),
            pl.BlockSpec((BR, D), lambda i: (i, 0)),
            pl.BlockSpec((D, D), lambda i: (0, 0)),
        ],
        out_specs=pl.BlockSpec((BR, D), lambda i: (i, 0)),
        out_shape=jax.ShapeDtypeStruct((N_NODES, D), jnp.float32),
    )(eps, x, p0, p1, wt)


def kernel(x, edge_index, edge_mask, W, eps):
    src = edge_index[0]
    dst = edge_index[1]
    partials = _sc_segment_partials(x, src, dst, edge_mask)
    return _tc_finish(x, partials[0], partials[1], W.T, eps)


# speedup vs baseline: 7.0290x; 1.3480x over previous
import functools

import jax
import jax.numpy as jnp
from jax import lax
from jax.experimental import pallas as pl
from jax.experimental.pallas import tpu as pltpu
from jax.experimental.pallas import tpu_sc as plsc

N_NODES = 10000
N_EDGES = 320000
D = 128

NC = 2
NS = 16
NW = NC * NS
CHUNK = 128
NCHUNKS = N_EDGES // CHUNK
ITERS = -(-NCHUNKS // NW)
ROWS_PER_SUB = 624
TAIL_BASE = ROWS_PER_SUB * NS
TAIL = N_NODES - TAIL_BASE
ZSIZES = (128, 128, 128, 128, 112)


def _sc_segment_partials(x, src, dst, mask):
    mesh = plsc.VectorSubcoreMesh(
        core_axis_name="c", subcore_axis_name="s", num_cores=NC, num_subcores=NS
    )

    @functools.partial(
        pl.kernel,
        out_type=jax.ShapeDtypeStruct((NC, N_NODES, D), jnp.float32),
        mesh=mesh,
        scratch_types=[
            pltpu.VMEM((CHUNK,), jnp.int32),
            pltpu.VMEM((CHUNK,), jnp.int32),
            pltpu.VMEM((CHUNK,), jnp.int32),
            pltpu.VMEM((CHUNK,), jnp.int32),
            pltpu.VMEM((CHUNK,), jnp.float32),
            pltpu.VMEM((CHUNK,), jnp.float32),
            pltpu.VMEM((CHUNK, D), jnp.float32),
            pltpu.VMEM((CHUNK, D), jnp.float32),
            pltpu.VMEM_SHARED((N_NODES, D), jnp.float32),
            pltpu.SemaphoreType.DMA,
            pltpu.SemaphoreType.DMA,
        ],
    )
    def body(x_hbm, src_hbm, dst_hbm, mask_hbm, out_hbm,
             src_a, src_b, dst_a, dst_b, mask_a, mask_b, rows_a, rows_b,
             accum, sem_a, sem_b):
        cid = lax.axis_index("c")
        sid = lax.axis_index("s")
        wid = sid * NC + cid

        zeros16 = jnp.zeros((16,), jnp.float32)

        def zero_row(r, _):
            for j in range(D // 16):
                rows_a[r, pl.ds(j * 16, 16)] = zeros16
            return _

        lax.fori_loop(0, CHUNK, zero_row, None)
        my_base = pl.multiple_of(sid * ROWS_PER_SUB, 8)
        off = 0
        for zsz in ZSIZES:
            pltpu.sync_copy(
                rows_a.at[pl.ds(0, zsz)],
                accum.at[pl.ds(pl.multiple_of(my_base + off, 8), zsz)],
            )
            off += zsz

        @pl.when(sid == 0)
        def _():
            pltpu.sync_copy(rows_a.at[pl.ds(0, TAIL)], accum.at[pl.ds(TAIL_BASE, TAIL)])

        plsc.subcore_barrier()

        def make_scale(mask_v, rows_v):
            def scale_group(g, _):
                mvec = mask_v[pl.ds(g * 16, 16)]
                for j in range(16):
                    m = jnp.full((16,), mvec[j], jnp.float32)
                    e = g * 16 + j
                    for k in range(D // 16):
                        rows_v[e, pl.ds(k * 16, 16)] = rows_v[e, pl.ds(k * 16, 16)] * m
                return _
            return scale_group

        def issue(it, src_v, rows_v, sem):
            chunk = it * NW + wid

            @pl.when(chunk < NCHUNKS)
            def _():
                base = pl.multiple_of(chunk * CHUNK, CHUNK)
                pltpu.sync_copy(src_hbm.at[pl.ds(base, CHUNK)], src_v)
                pltpu.async_copy(x_hbm.at[src_v], rows_v, sem)

        def process(it, src_v, dst_v, mask_v, rows_v, sem):
            chunk = it * NW + wid

            @pl.when(chunk < NCHUNKS)
            def _():
                base = pl.multiple_of(chunk * CHUNK, CHUNK)
                pltpu.sync_copy(dst_hbm.at[pl.ds(base, CHUNK)], dst_v)
                pltpu.sync_copy(mask_hbm.at[pl.ds(base, CHUNK)], mask_v)
                pltpu.make_async_copy(x_hbm.at[src_v], rows_v, sem).wait()
                lax.fori_loop(0, CHUNK // 16, make_scale(mask_v, rows_v), None)
                pltpu.sync_copy(rows_v, accum.at[dst_v], add=True)

        issue(0, src_a, rows_a, sem_a)

        def pair_body(i, _):
            it0 = i * 2
            issue(it0 + 1, src_b, rows_b, sem_b)
            process(it0, src_a, dst_a, mask_a, rows_a, sem_a)
            issue(it0 + 2, src_a, rows_a, sem_a)
            process(it0 + 1, src_b, dst_b, mask_b, rows_b, sem_b)
            return _

        lax.fori_loop(0, (ITERS + 1) // 2, pair_body, None)
        plsc.subcore_barrier()

        pltpu.sync_copy(
            accum.at[pl.ds(my_base, ROWS_PER_SUB)],
            out_hbm.at[cid, pl.ds(my_base, ROWS_PER_SUB)],
        )

        @pl.when(sid == 0)
        def _():
            pltpu.sync_copy(
                accum.at[pl.ds(TAIL_BASE, TAIL)],
                out_hbm.at[cid, pl.ds(TAIL_BASE, TAIL)],
            )

    return body(x, src, dst, mask)


def _tc_finish(x, p0, p1, wt, eps):
    BR = 1000

    def body(eps_ref, x_ref, p0_ref, p1_ref, wt_ref, out_ref):
        h = (1.0 + eps_ref[0]) * x_ref[...] + p0_ref[...] + p1_ref[...]
        out_ref[...] = jnp.dot(h, wt_ref[...], preferred_element_type=jnp.float32)

    return pl.pallas_call(
        body,
        grid=(N_NODES // BR,),
        in_specs=[
            pl.BlockSpec(memory_space=pltpu.SMEM),
            pl.BlockSpec((BR, D), lambda i: (i, 0)),
            pl.BlockSpec((BR, D), lambda i: (i, 0)),
            pl.BlockSpec((BR, D), lambda i: (i, 0)),
            pl.BlockSpec((D, D), lambda i: (0, 0)),
        ],
        out_specs=pl.BlockSpec((BR, D), lambda i: (i, 0)),
        out_shape=jax.ShapeDtypeStruct((N_NODES, D), jnp.float32),
    )(eps, x, p0, p1, wt)


def kernel(x, edge_index, edge_mask, W, eps):
    src = edge_index[0]
    dst = edge_index[1]
    partials = _sc_segment_partials(x, src, dst, edge_mask)
    return _tc_finish(x, partials[0], partials[1], W.T, eps)
